# baseline (speedup 1.0000x reference)
import functools

import jax
import jax.numpy as jnp
from jax.experimental import pallas as pl
from jax.experimental.pallas import tpu as pltpu

_B, _S, _D, _E, _TOPK = 4, 2048, 2048, 16, 2


def _router_block(x_ref, w_ref, b_ref, n_ref, probs_ref, idx_ref, loss_ref,
                  acc_ref, *, n_tokens):
    i = pl.program_id(0)
    nb = pl.num_programs(0)
    y = jnp.dot(x_ref[:, :], w_ref[:, :], preferred_element_type=jnp.float32)
    y = y + b_ref[:, :]
    logits = y[:, :_E]
    noise_logits = y[:, _E:]
    sp = jnp.maximum(noise_logits, 0.0) + jnp.log1p(jnp.exp(-jnp.abs(noise_logits)))
    noisy = logits + n_ref[:, :] * sp

    iota = jax.lax.broadcasted_iota(jnp.int32, noisy.shape, 1)
    m1 = jnp.max(noisy, axis=1, keepdims=True)
    i1 = jnp.min(jnp.where(noisy == m1, iota, _E), axis=1, keepdims=True)
    masked = jnp.where(iota == i1, -jnp.inf, noisy)
    m2 = jnp.max(masked, axis=1, keepdims=True)
    i2 = jnp.min(jnp.where(masked == m2, iota, _E), axis=1, keepdims=True)

    mask = (iota == i1) | (iota == i2)
    p = jnp.where(mask, jnp.exp(noisy - m1), 0.0)
    probs = p / jnp.sum(p, axis=1, keepdims=True)
    probs_ref[:, :] = probs
    idx_ref[:, :] = jnp.concatenate([i1, i2], axis=1)

    bp = jnp.sum(probs, axis=0, keepdims=True)
    bm = jnp.sum(mask.astype(jnp.float32), axis=0, keepdims=True)

    @pl.when(i == 0)
    def _():
        acc_ref[0:1, :] = bp
        acc_ref[1:2, :] = bm

    @pl.when(i > 0)
    def _():
        acc_ref[0:1, :] = acc_ref[0:1, :] + bp
        acc_ref[1:2, :] = acc_ref[1:2, :] + bm

    @pl.when(i == nb - 1)
    def _():
        scale = jnp.float32(_E) / jnp.float32(n_tokens * n_tokens)
        loss = scale * jnp.sum(acc_ref[0:1, :] * acc_ref[1:2, :],
                               axis=1, keepdims=True)
        loss_ref[:, :] = loss


def kernel(mh_output, W_route, b_route, W_noise, b_noise):
    n_tokens = _B * _S
    x = mh_output.reshape(n_tokens, _D)
    w = jnp.concatenate([W_route, W_noise], axis=1)
    b = jnp.concatenate([b_route, b_noise], axis=0).reshape(1, 2 * _E)
    noise = jax.random.normal(jax.random.key(42), (n_tokens, _E), dtype=jnp.float32)

    T = 1024
    nb = n_tokens // T
    probs, idx, loss = pl.pallas_call(
        functools.partial(_router_block, n_tokens=n_tokens),
        grid=(nb,),
        in_specs=[
            pl.BlockSpec((T, _D), lambda i: (i, 0)),
            pl.BlockSpec((_D, 2 * _E), lambda i: (0, 0)),
            pl.BlockSpec((1, 2 * _E), lambda i: (0, 0)),
            pl.BlockSpec((T, _E), lambda i: (i, 0)),
        ],
        out_specs=[
            pl.BlockSpec((T, _E), lambda i: (i, 0)),
            pl.BlockSpec((T, _TOPK), lambda i: (i, 0)),
            pl.BlockSpec((1, 1), lambda i: (0, 0)),
        ],
        out_shape=[
            jax.ShapeDtypeStruct((n_tokens, _E), jnp.float32),
            jax.ShapeDtypeStruct((n_tokens, _TOPK), jnp.int32),
            jax.ShapeDtypeStruct((1, 1), jnp.float32),
        ],
        scratch_shapes=[pltpu.VMEM((2, _E), jnp.float32)],
    )(x, w, b, noise)

    return (probs.reshape(_B, _S, _E),
            idx.reshape(_B, _S, _TOPK),
            loss.reshape(()))


# speedup vs baseline: 2.0763x; 2.0763x over previous
import functools

import jax
import jax.numpy as jnp
from jax.experimental import pallas as pl
from jax.experimental.pallas import tpu as pltpu

_B, _S, _D, _E, _TOPK = 4, 2048, 2048, 16, 2


def _router_block(x_ref, w_ref, b_ref, n_ref, probs_ref, idx_ref, loss_ref,
                  acc_ref, *, n_tokens):
    i = pl.program_id(0)
    nb = pl.num_programs(0)
    y = jnp.dot(x_ref[:, :], w_ref[:, :], preferred_element_type=jnp.float32)
    yt = y.T + b_ref[:, :]
    logits = yt[:_E, :]
    noise_logits = yt[_E:, :]
    sp = jnp.maximum(noise_logits, 0.0) + jnp.log1p(jnp.exp(-jnp.abs(noise_logits)))
    noisy = logits + n_ref[:, :] * sp

    iota = jax.lax.broadcasted_iota(jnp.int32, noisy.shape, 0)
    m1 = jnp.max(noisy, axis=0, keepdims=True)
    i1 = jnp.min(jnp.where(noisy == m1, iota, _E), axis=0, keepdims=True)
    masked = jnp.where(iota == i1, -jnp.inf, noisy)
    m2 = jnp.max(masked, axis=0, keepdims=True)
    i2 = jnp.min(jnp.where(masked == m2, iota, _E), axis=0, keepdims=True)

    mask = (iota == i1) | (iota == i2)
    p = jnp.where(mask, jnp.exp(noisy - m1), 0.0)
    probs = p / jnp.sum(p, axis=0, keepdims=True)
    probs_ref[:, :] = probs
    idx_ref[:, :] = jnp.concatenate([i1, i2], axis=0)

    bp = jnp.sum(probs, axis=1, keepdims=True)
    bm = jnp.sum(mask.astype(jnp.float32), axis=1, keepdims=True)

    @pl.when(i == 0)
    def _():
        acc_ref[:, 0:1] = bp
        acc_ref[:, 1:2] = bm

    @pl.when(i > 0)
    def _():
        acc_ref[:, 0:1] = acc_ref[:, 0:1] + bp
        acc_ref[:, 1:2] = acc_ref[:, 1:2] + bm

    @pl.when(i == nb - 1)
    def _():
        scale = jnp.float32(_E) / jnp.float32(n_tokens * n_tokens)
        loss = scale * jnp.sum(acc_ref[:, 0:1] * acc_ref[:, 1:2],
                               axis=0, keepdims=True)
        loss_ref[:, :] = loss


def kernel(mh_output, W_route, b_route, W_noise, b_noise):
    n_tokens = _B * _S
    x = mh_output.reshape(n_tokens, _D)
    w = jnp.concatenate([W_route, W_noise], axis=1)
    b = jnp.concatenate([b_route, b_noise], axis=0).reshape(2 * _E, 1)
    with jax.ensure_compile_time_eval():
        noise = jax.random.normal(jax.random.key(42), (_B, _S, _E),
                                  dtype=jnp.float32)
        noise_t = noise.reshape(n_tokens, _E).T.copy()

    T = 512
    nb = n_tokens // T
    probs_t, idx_t, loss = pl.pallas_call(
        functools.partial(_router_block, n_tokens=n_tokens),
        grid=(nb,),
        in_specs=[
            pl.BlockSpec((T, _D), lambda i: (i, 0)),
            pl.BlockSpec((_D, 2 * _E), lambda i: (0, 0)),
            pl.BlockSpec((2 * _E, 1), lambda i: (0, 0)),
            pl.BlockSpec((_E, T), lambda i: (0, i)),
        ],
        out_specs=[
            pl.BlockSpec((_E, T), lambda i: (0, i)),
            pl.BlockSpec((_TOPK, T), lambda i: (0, i)),
            pl.BlockSpec((1, 1), lambda i: (0, 0)),
        ],
        out_shape=[
            jax.ShapeDtypeStruct((_E, n_tokens), jnp.float32),
            jax.ShapeDtypeStruct((_TOPK, n_tokens), jnp.int32),
            jax.ShapeDtypeStruct((1, 1), jnp.float32),
        ],
        scratch_shapes=[pltpu.VMEM((_E, 2), jnp.float32)],
    )(x, w, b, noise_t)

    return (probs_t.T.reshape(_B, _S, _E),
            idx_t.T.reshape(_B, _S, _TOPK),
            loss.reshape(()))


# speedup vs baseline: 2.4799x; 1.1944x over previous
import functools

import jax
import jax.numpy as jnp
from jax.experimental import pallas as pl
from jax.experimental.pallas import tpu as pltpu

_B, _S, _D, _E, _TOPK = 4, 2048, 2048, 16, 2


def _router_block(x_ref, w_ref, b_ref, n_ref, probs_ref, idx_ref, loss_ref,
                  acc_ref, *, n_tokens):
    i = pl.program_id(0)
    nb = pl.num_programs(0)
    y = jnp.dot(x_ref[:, :], w_ref[:, :], preferred_element_type=jnp.float32)
    yt = y.T + b_ref[:, :]
    logits = yt[:_E, :]
    noise_logits = yt[_E:, :]
    sp = jnp.maximum(noise_logits, 0.0) + jnp.log1p(jnp.exp(-jnp.abs(noise_logits)))
    noisy = logits + n_ref[:, :] * sp

    iota = jax.lax.broadcasted_iota(jnp.int32, noisy.shape, 0)
    m1 = jnp.max(noisy, axis=0, keepdims=True)
    i1 = jnp.min(jnp.where(noisy == m1, iota, _E), axis=0, keepdims=True)
    masked = jnp.where(iota == i1, -jnp.inf, noisy)
    m2 = jnp.max(masked, axis=0, keepdims=True)
    i2 = jnp.min(jnp.where(masked == m2, iota, _E), axis=0, keepdims=True)

    mask = (iota == i1) | (iota == i2)
    p = jnp.where(mask, jnp.exp(noisy - m1), 0.0)
    probs = p / jnp.sum(p, axis=0, keepdims=True)
    probs_ref[:, :] = probs
    idx_ref[:, :] = jnp.concatenate([i1, i2], axis=0)

    bp = jnp.sum(probs, axis=1, keepdims=True)
    bm = jnp.sum(mask.astype(jnp.float32), axis=1, keepdims=True)

    @pl.when(i == 0)
    def _():
        acc_ref[:, 0:1] = bp
        acc_ref[:, 1:2] = bm

    @pl.when(i > 0)
    def _():
        acc_ref[:, 0:1] = acc_ref[:, 0:1] + bp
        acc_ref[:, 1:2] = acc_ref[:, 1:2] + bm

    @pl.when(i == nb - 1)
    def _():
        scale = jnp.float32(_E) / jnp.float32(n_tokens * n_tokens)
        loss = scale * jnp.sum(acc_ref[:, 0:1] * acc_ref[:, 1:2],
                               axis=0, keepdims=True)
        loss_ref[:, :] = loss


def kernel(mh_output, W_route, b_route, W_noise, b_noise):
    n_tokens = _B * _S
    x = mh_output.reshape(n_tokens, _D)
    w = jnp.concatenate([W_route, W_noise], axis=1)
    b = jnp.concatenate([b_route, b_noise], axis=0).reshape(2 * _E, 1)
    with jax.ensure_compile_time_eval():
        noise = jax.random.normal(jax.random.key(42), (_B, _S, _E),
                                  dtype=jnp.float32)
        noise_t = noise.reshape(n_tokens, _E).T.copy()

    T = 1024
    nb = n_tokens // T
    probs_t, idx_t, loss = pl.pallas_call(
        functools.partial(_router_block, n_tokens=n_tokens),
        grid=(nb,),
        in_specs=[
            pl.BlockSpec((T, _D), lambda i: (i, 0)),
            pl.BlockSpec((_D, 2 * _E), lambda i: (0, 0)),
            pl.BlockSpec((2 * _E, 1), lambda i: (0, 0)),
            pl.BlockSpec((_E, T), lambda i: (0, i)),
        ],
        out_specs=[
            pl.BlockSpec((_E, T), lambda i: (0, i)),
            pl.BlockSpec((_TOPK, T), lambda i: (0, i)),
            pl.BlockSpec((1, 1), lambda i: (0, 0)),
        ],
        out_shape=[
            jax.ShapeDtypeStruct((_E, n_tokens), jnp.float32),
            jax.ShapeDtypeStruct((_TOPK, n_tokens), jnp.int32),
            jax.ShapeDtypeStruct((1, 1), jnp.float32),
        ],
        scratch_shapes=[pltpu.VMEM((_E, 2), jnp.float32)],
    )(x, w, b, noise_t)

    return (probs_t.T.reshape(_B, _S, _E),
            idx_t.T.reshape(_B, _S, _TOPK),
            loss.reshape(()))
